# Initial kernel scaffold; baseline (speedup 1.0000x reference)
import jax
import jax.numpy as jnp
from jax import lax
from jax.experimental import pallas as pl
from jax.experimental.pallas import tpu as pltpu
from jax.experimental.pallas import tpu_sc as plsc

HOTN = 32768
HASH_SIZE = 500000
EMB_DIM = 32
BATCH = 16384

_NC = 2
_NS = 16
_NW = _NC * _NS
_BPW = BATCH // _NW
_NVEC = _BPW // 16
_GCH = 128
_NG = _BPW // _GCH


def _sc_body(inp_hbm, wh_hbm, whash_hbm, out_hbm,
             raw_v, idxh_v, idxc_v, maskf_v, rows_hot, rows_cold, sem):
    wid = lax.axis_index("s") * _NC + lax.axis_index("c")
    base = wid * _BPW

    pltpu.sync_copy(inp_hbm.at[pl.ds(base, _BPW)], raw_v)

    for i in range(_NVEC):
        v = raw_v[pl.ds(i * 16, 16)]
        s = (v & 31) + ((v >> 5) & 31) + ((v >> 10) & 31) + ((v >> 15) & 31)
        s = (s & 31) + (s >> 5)
        hot = jnp.logical_or(s == 0, s == 31)
        maskf_v[pl.ds(i * 16, 16)] = jnp.where(hot, 1.0, 0.0).astype(jnp.float32)
        idxh_v[i // 8, pl.ds((i % 8) * 16, 16)] = v & (HOTN - 1)
        idxc_v[i // 8, pl.ds((i % 8) * 16, 16)] = jnp.where(
            v >= HASH_SIZE, v - HASH_SIZE, v)

    copies = []
    for j in range(_NG):
        copies.append(pltpu.async_copy(
            wh_hbm.at[idxh_v.at[j]], rows_hot.at[pl.ds(j * _GCH, _GCH)], sem))
        copies.append(pltpu.async_copy(
            whash_hbm.at[idxc_v.at[j]], rows_cold.at[pl.ds(j * _GCH, _GCH)], sem))
    for c in copies:
        c.wait()

    def body(i, _):
        m = maskf_v[i]
        for c0 in (0, 16):
            h = rows_hot[i, pl.ds(c0, 16)]
            g = rows_cold[i, pl.ds(c0, 16)]
            rows_hot[i, pl.ds(c0, 16)] = g + m * (h - g)
        return 0

    lax.fori_loop(0, _BPW, body, 0)

    pltpu.sync_copy(rows_hot, out_hbm.at[pl.ds(base, _BPW)])


@jax.jit
def _run(inp, wh, whash):
    mesh = plsc.VectorSubcoreMesh(core_axis_name="c", subcore_axis_name="s")
    f = pl.kernel(
        _sc_body,
        out_type=jax.ShapeDtypeStruct((BATCH, EMB_DIM), jnp.float32),
        mesh=mesh,
        scratch_types=[
            pltpu.VMEM((_BPW,), jnp.int32),
            pltpu.VMEM((_NG, _GCH), jnp.int32),
            pltpu.VMEM((_NG, _GCH), jnp.int32),
            pltpu.VMEM((_BPW,), jnp.float32),
            pltpu.VMEM((_BPW, EMB_DIM), jnp.float32),
            pltpu.VMEM((_BPW, EMB_DIM), jnp.float32),
            pltpu.SemaphoreType.DMA,
        ],
    )
    return f(inp, wh, whash)


def kernel(input, offsets, weight_h, weight_hash):
    del offsets
    return _run(input.astype(jnp.int32), weight_h, weight_hash)


# speedup vs baseline: 2.5181x; 2.5181x over previous
import jax
import jax.numpy as jnp
from jax import lax
from jax.experimental import pallas as pl
from jax.experimental.pallas import tpu as pltpu
from jax.experimental.pallas import tpu_sc as plsc

HOTN = 32768
HASH_SIZE = 500000
EMB_DIM = 32
BATCH = 16384

_NC = 2
_NS = 16
_NW = _NC * _NS
_BPW = BATCH // _NW
_NVEC = _BPW // 16
_GCH = 128
_NG = _BPW // _GCH


def _sc_body(inp_hbm, wh_hbm, whash_hbm, out_hbm,
             raw_v, idxh_v, idxc_v, maskf_v, rows_hot, rows_cold, sem):
    wid = lax.axis_index("s") * _NC + lax.axis_index("c")
    base = wid * _BPW

    pltpu.sync_copy(inp_hbm.at[pl.ds(base, _BPW)], raw_v)

    for i in range(_NVEC):
        v = raw_v[pl.ds(i * 16, 16)]
        s = (v & 31) + ((v >> 5) & 31) + ((v >> 10) & 31) + ((v >> 15) & 31)
        s = (s & 31) + (s >> 5)
        hot = jnp.logical_or(s == 0, s == 31)
        maskf_v[pl.ds(i * 16, 16)] = jnp.where(hot, 1.0, 0.0).astype(jnp.float32)
        idxh_v[i // 8, pl.ds((i % 8) * 16, 16)] = v & (HOTN - 1)
        idxc_v[i // 8, pl.ds((i % 8) * 16, 16)] = jnp.where(
            v >= HASH_SIZE, v - HASH_SIZE, v)

    copies = []
    for j in range(_NG):
        copies.append(pltpu.async_copy(
            wh_hbm.at[idxh_v.at[j]], rows_hot.at[pl.ds(j * _GCH, _GCH)], sem))
        copies.append(pltpu.async_copy(
            whash_hbm.at[idxc_v.at[j]], rows_cold.at[pl.ds(j * _GCH, _GCH)], sem))
    for c in copies:
        c.wait()

    def body(blk, _):
        m16 = maskf_v[pl.ds(blk * 16, 16)]
        for j in range(16):
            m = m16[j]
            r = blk * 16 + j
            for c0 in (0, 16):
                h = rows_hot[r, pl.ds(c0, 16)]
                g = rows_cold[r, pl.ds(c0, 16)]
                rows_hot[r, pl.ds(c0, 16)] = g + m * (h - g)
        return 0

    lax.fori_loop(0, _NVEC, body, 0)

    pltpu.sync_copy(rows_hot, out_hbm.at[pl.ds(base, _BPW)])


@jax.jit
def _run(inp, wh, whash):
    mesh = plsc.VectorSubcoreMesh(core_axis_name="c", subcore_axis_name="s")
    f = pl.kernel(
        _sc_body,
        out_type=jax.ShapeDtypeStruct((BATCH, EMB_DIM), jnp.float32),
        mesh=mesh,
        compiler_params=pltpu.CompilerParams(use_tc_tiling_on_sc=False),
        scratch_types=[
            pltpu.VMEM((_BPW,), jnp.int32),
            pltpu.VMEM((_NG, _GCH), jnp.int32),
            pltpu.VMEM((_NG, _GCH), jnp.int32),
            pltpu.VMEM((_BPW,), jnp.float32),
            pltpu.VMEM((_BPW, EMB_DIM), jnp.float32),
            pltpu.VMEM((_BPW, EMB_DIM), jnp.float32),
            pltpu.SemaphoreType.DMA,
        ],
    )
    return f(inp, wh, whash)


def kernel(input, offsets, weight_h, weight_hash):
    del offsets
    return _run(input.astype(jnp.int32), weight_h, weight_hash)
